# Initial kernel scaffold; baseline (speedup 1.0000x reference)
import jax
import jax.numpy as jnp
from jax.experimental import pallas as pl


def kernel(x):
    raise NotImplementedError("write your pallas kernel here")


# speedup vs baseline: 125.9594x; 125.9594x over previous
import functools

import jax
import jax.numpy as jnp
from jax.experimental import pallas as pl
from jax.experimental.pallas import tpu as pltpu

STRICT_BONUS = 10.0
NOISE_EPS = 1e-07


def _shift(plane, dh, dw):
    v = plane
    if dh == -1:
        v = jnp.concatenate([v[:1, :], v[:-1, :]], axis=0)
    elif dh == 1:
        v = jnp.concatenate([v[1:, :], v[-1:, :]], axis=0)
    if dw == -1:
        v = jnp.concatenate([v[:, :1], v[:, :-1]], axis=1)
    elif dw == 1:
        v = jnp.concatenate([v[:, 1:], v[:, -1:]], axis=1)
    return v


def _stencil_kernel(x_ref, noise_ref, coords_ref, y_ref, *, B, D, H, W):
    n00 = noise_ref[0, 0]; n01 = noise_ref[0, 1]; n02 = noise_ref[0, 2]
    n10 = noise_ref[1, 0]; n11 = noise_ref[1, 1]; n12 = noise_ref[1, 2]
    n20 = noise_ref[2, 0]; n21 = noise_ref[2, 1]; n22 = noise_ref[2, 2]

    row_f = jax.lax.broadcasted_iota(jnp.int32, (H, W), 0).astype(jnp.float32)
    col_f = jax.lax.broadcasted_iota(jnp.int32, (H, W), 1).astype(jnp.float32)

    for b in range(B):
        planes = [x_ref[b, d] for d in range(D)]
        for d in range(D):
            z0 = planes[d]
            zm = planes[max(d - 1, 0)]
            zp = planes[min(d + 1, D - 1)]

            s0 = {(dh, dw): _shift(z0, dh, dw) for dh in (-1, 0, 1) for dw in (-1, 0, 1)}
            sm = {(dh, dw): _shift(zm, dh, dw) for dh in (-1, 0, 1) for dw in (-1, 0, 1)}
            sp = {(dh, dw): _shift(zp, dh, dw) for dh in (-1, 0, 1) for dw in (-1, 0, 1)}

            gx = 0.5 * (s0[(0, 1)] - s0[(0, -1)])
            gy = 0.5 * (s0[(1, 0)] - s0[(-1, 0)])
            gs = 0.5 * (zp - zm)
            dxx = s0[(0, 1)] - 2.0 * z0 + s0[(0, -1)]
            dyy = s0[(1, 0)] - 2.0 * z0 + s0[(-1, 0)]
            dss = zp - 2.0 * z0 + zm
            dxy = 0.25 * (s0[(1, 1)] - s0[(1, -1)] - s0[(-1, 1)] + s0[(-1, -1)])
            dys = 0.25 * (sp[(1, 0)] - sp[(-1, 0)] - sm[(1, 0)] + sm[(-1, 0)])
            dxs = 0.25 * (sp[(0, 1)] - sp[(0, -1)] - sm[(0, 1)] + sm[(0, -1)])

            mx = None
            for key in s0:
                for s in (sm, sp):
                    v = s[key]
                    mx = v if mx is None else jnp.maximum(mx, v)
                if key != (0, 0):
                    mx = jnp.maximum(mx, s0[key])
            mask = z0 > mx

            zero = jnp.zeros_like(z0)
            one = jnp.ones_like(z0)
            ha = jnp.where(mask, dxx + n00, one)
            hb = jnp.where(mask, dxy + n01, zero)
            hc = jnp.where(mask, dxs + n02, zero)
            hd = jnp.where(mask, dxy + n10, zero)
            he = jnp.where(mask, dyy + n11, one)
            hf = jnp.where(mask, dys + n12, zero)
            hg = jnp.where(mask, dxs + n20, zero)
            hh = jnp.where(mask, dys + n21, zero)
            hi = jnp.where(mask, dss + n22, one)

            A11 = he * hi - hf * hh; A12 = hc * hh - hb * hi; A13 = hb * hf - hc * he
            A21 = hf * hg - hd * hi; A22 = ha * hi - hc * hg; A23 = hc * hd - ha * hf
            A31 = hd * hh - he * hg; A32 = hb * hg - ha * hh; A33 = ha * he - hb * hd
            det = ha * A11 + hb * A21 + hc * A31
            inv_det = 1.0 / det
            sx = (A11 * inv_det) * gx + (A12 * inv_det) * gy + (A13 * inv_det) * gs
            sy = (A21 * inv_det) * gx + (A22 * inv_det) * gy + (A23 * inv_det) * gs
            ss = (A31 * inv_det) * gx + (A32 * inv_det) * gy + (A33 * inv_det) * gs

            dx0 = jnp.where(mask, -sx, 0.0)
            dx1 = jnp.where(mask, -sy, 0.0)
            dx2 = jnp.where(mask, -ss, 0.0)
            big = jnp.maximum(jnp.maximum(jnp.abs(dx0), jnp.abs(dx1)), jnp.abs(dx2)) > 0.7
            dx0 = jnp.where(big, 0.0, dx0)
            dx1 = jnp.where(big, 0.0, dx1)
            dx2 = jnp.where(big, 0.0, dx2)

            dy_corr = 0.5 * (gx * dx0 + gy * dx1 + gs * dx2)
            y_ref[b, 0, d] = z0 + dy_corr + STRICT_BONUS * mask.astype(jnp.float32)

            coords_ref[b, 0, 0, d] = float(d) + dx2
            coords_ref[b, 0, 1, d] = row_f + dx1
            coords_ref[b, 0, 2, d] = col_f + dx0


@jax.jit
def kernel(x):
    B, C, D, H, W = x.shape
    noise = jnp.abs(jax.random.uniform(jax.random.key(42), (3, 3), dtype=x.dtype)) * NOISE_EPS
    xr = x.reshape(B * C, D, H, W)
    coords, y = pl.pallas_call(
        functools.partial(_stencil_kernel, B=B * C, D=D, H=H, W=W),
        out_shape=(
            jax.ShapeDtypeStruct((B, C, 3, D, H, W), x.dtype),
            jax.ShapeDtypeStruct((B, C, D, H, W), x.dtype),
        ),
        in_specs=[
            pl.BlockSpec(memory_space=pltpu.VMEM),
            pl.BlockSpec(memory_space=pltpu.SMEM),
        ],
        out_specs=(
            pl.BlockSpec(memory_space=pltpu.VMEM),
            pl.BlockSpec(memory_space=pltpu.VMEM),
        ),
    )(xr, noise)
    return coords, y


# speedup vs baseline: 139.1969x; 1.1051x over previous
import functools

import jax
import jax.numpy as jnp
from jax.experimental import pallas as pl
from jax.experimental.pallas import tpu as pltpu

STRICT_BONUS = 10.0
NOISE_EPS = 1e-07


def _shift_h(v, dh):
    if dh == -1:
        return jnp.concatenate([v[:1, :], v[:-1, :]], axis=0)
    return jnp.concatenate([v[1:, :], v[-1:, :]], axis=0)


def _shift_w(v, dw):
    if dw == -1:
        return jnp.concatenate([v[:, :1], v[:, :-1]], axis=1)
    return jnp.concatenate([v[:, 1:], v[:, -1:]], axis=1)


class _Plane:
    def __init__(self, p):
        self.p = p
        self.hm = _shift_h(p, -1)
        self.hp = _shift_h(p, 1)
        self.wm = _shift_w(p, -1)
        self.wp = _shift_w(p, 1)
        self.d_mm = _shift_w(self.hm, -1)
        self.d_mp = _shift_w(self.hm, 1)
        self.d_pm = _shift_w(self.hp, -1)
        self.d_pp = _shift_w(self.hp, 1)
        vm = jnp.maximum(jnp.maximum(self.hm, self.hp), p)
        vmw_m = _shift_w(vm, -1)
        vmw_p = _shift_w(vm, 1)
        self.max8 = jnp.maximum(jnp.maximum(vmw_m, vmw_p),
                                jnp.maximum(self.hm, self.hp))
        self.max9 = jnp.maximum(self.max8, p)


def _stencil_kernel(x_ref, noise_ref, coords_ref, y_ref, *, D, H, W):
    n00 = noise_ref[0, 0]; n01 = noise_ref[0, 1]; n02 = noise_ref[0, 2]
    n10 = noise_ref[1, 0]; n11 = noise_ref[1, 1]; n12 = noise_ref[1, 2]
    n20 = noise_ref[2, 0]; n21 = noise_ref[2, 1]; n22 = noise_ref[2, 2]

    row_f = jax.lax.broadcasted_iota(jnp.int32, (H, W), 0).astype(jnp.float32)
    col_f = jax.lax.broadcasted_iota(jnp.int32, (H, W), 1).astype(jnp.float32)

    P = [_Plane(x_ref[0, d]) for d in range(D)]

    for d in range(D):
        c = P[d]
        lo = P[max(d - 1, 0)]
        hi = P[min(d + 1, D - 1)]
        z0 = c.p

        gx = 0.5 * (c.wp - c.wm)
        gy = 0.5 * (c.hp - c.hm)
        gs = 0.5 * (hi.p - lo.p)
        dxx = c.wp + c.wm - 2.0 * z0
        dyy = c.hp + c.hm - 2.0 * z0
        dss = hi.p + lo.p - 2.0 * z0
        dxy = 0.25 * ((c.d_pp - c.d_pm) - (c.d_mp - c.d_mm))
        dys = 0.25 * ((hi.hp - hi.hm) - (lo.hp - lo.hm))
        dxs = 0.25 * ((hi.wp - hi.wm) - (lo.wp - lo.wm))

        mx = jnp.maximum(jnp.maximum(lo.max9, hi.max9), c.max8)
        mask = z0 > mx

        zero = jnp.zeros_like(z0)
        one = jnp.ones_like(z0)
        ha = jnp.where(mask, dxx + n00, one)
        hb = jnp.where(mask, dxy + n01, zero)
        hc = jnp.where(mask, dxs + n02, zero)
        hd = jnp.where(mask, dxy + n10, zero)
        he = jnp.where(mask, dyy + n11, one)
        hf = jnp.where(mask, dys + n12, zero)
        hg = jnp.where(mask, dxs + n20, zero)
        hh = jnp.where(mask, dys + n21, zero)
        hi_ = jnp.where(mask, dss + n22, one)

        A11 = he * hi_ - hf * hh; A12 = hc * hh - hb * hi_; A13 = hb * hf - hc * he
        A21 = hf * hg - hd * hi_; A22 = ha * hi_ - hc * hg; A23 = hc * hd - ha * hf
        A31 = hd * hh - he * hg; A32 = hb * hg - ha * hh; A33 = ha * he - hb * hd
        det = ha * A11 + hb * A21 + hc * A31
        inv_det = 1.0 / det
        sx = (A11 * inv_det) * gx + (A12 * inv_det) * gy + (A13 * inv_det) * gs
        sy = (A21 * inv_det) * gx + (A22 * inv_det) * gy + (A23 * inv_det) * gs
        ss = (A31 * inv_det) * gx + (A32 * inv_det) * gy + (A33 * inv_det) * gs

        dx0 = jnp.where(mask, -sx, 0.0)
        dx1 = jnp.where(mask, -sy, 0.0)
        dx2 = jnp.where(mask, -ss, 0.0)
        big = jnp.maximum(jnp.maximum(jnp.abs(dx0), jnp.abs(dx1)), jnp.abs(dx2)) > 0.7
        dx0 = jnp.where(big, 0.0, dx0)
        dx1 = jnp.where(big, 0.0, dx1)
        dx2 = jnp.where(big, 0.0, dx2)

        dy_corr = 0.5 * (gx * dx0 + gy * dx1 + gs * dx2)
        y_ref[0, 0, d] = z0 + dy_corr + STRICT_BONUS * mask.astype(jnp.float32)

        coords_ref[0, 0, 0, d] = float(d) + dx2
        coords_ref[0, 0, 1, d] = row_f + dx1
        coords_ref[0, 0, 2, d] = col_f + dx0


@jax.jit
def kernel(x):
    B, C, D, H, W = x.shape
    noise = jnp.abs(jax.random.uniform(jax.random.key(42), (3, 3), dtype=x.dtype)) * NOISE_EPS
    xr = x.reshape(B * C, D, H, W)
    coords, y = pl.pallas_call(
        functools.partial(_stencil_kernel, D=D, H=H, W=W),
        grid=(B * C,),
        out_shape=(
            jax.ShapeDtypeStruct((B, C, 3, D, H, W), x.dtype),
            jax.ShapeDtypeStruct((B, C, D, H, W), x.dtype),
        ),
        in_specs=[
            pl.BlockSpec((1, D, H, W), lambda b: (b, 0, 0, 0)),
            pl.BlockSpec(memory_space=pltpu.SMEM),
        ],
        out_specs=(
            pl.BlockSpec((1, 1, 3, D, H, W), lambda b: (b, 0, 0, 0, 0, 0)),
            pl.BlockSpec((1, 1, D, H, W), lambda b: (b, 0, 0, 0, 0)),
        ),
        compiler_params=pltpu.CompilerParams(
            dimension_semantics=("arbitrary",),
        ),
    )(xr, noise)
    return coords, y


# speedup vs baseline: 242.2681x; 1.7405x over previous
import functools

import jax
import jax.numpy as jnp
from jax.experimental import pallas as pl
from jax.experimental.pallas import tpu as pltpu

STRICT_BONUS = 10.0
NOISE_EPS = 1e-07


def _shift_h(v, dh):
    if dh == -1:
        return jnp.concatenate([v[:1, :], v[:-1, :]], axis=0)
    return jnp.concatenate([v[1:, :], v[-1:, :]], axis=0)


def _shift_w(v, dw):
    if dw == -1:
        return jnp.concatenate([v[:, :1], v[:, :-1]], axis=1)
    return jnp.concatenate([v[:, 1:], v[:, -1:]], axis=1)


class _Plane:
    def __init__(self, p, need_diag, need_max8, need_max9):
        self.p = p
        self.hm = _shift_h(p, -1)
        self.hp = _shift_h(p, 1)
        self.wm = _shift_w(p, -1)
        self.wp = _shift_w(p, 1)
        if need_diag:
            self.d_mm = _shift_w(self.hm, -1)
            self.d_mp = _shift_w(self.hm, 1)
            self.d_pm = _shift_w(self.hp, -1)
            self.d_pp = _shift_w(self.hp, 1)
        if need_max8 or need_max9:
            vm = jnp.maximum(jnp.maximum(self.hm, self.hp), p)
            vmw_m = _shift_w(vm, -1)
            vmw_p = _shift_w(vm, 1)
            self.max8 = jnp.maximum(jnp.maximum(vmw_m, vmw_p),
                                    jnp.maximum(self.hm, self.hp))
            if need_max9:
                self.max9 = jnp.maximum(self.max8, p)


def _stencil_kernel(x_ref, noise_ref, coords_ref, y_ref, *, D, H, W):
    n00 = noise_ref[0, 0]; n01 = noise_ref[0, 1]; n02 = noise_ref[0, 2]
    n10 = noise_ref[1, 0]; n11 = noise_ref[1, 1]; n12 = noise_ref[1, 2]
    n20 = noise_ref[2, 0]; n21 = noise_ref[2, 1]; n22 = noise_ref[2, 2]

    row_f = jax.lax.broadcasted_iota(jnp.int32, (H, W), 0).astype(jnp.float32)
    col_f = jax.lax.broadcasted_iota(jnp.int32, (H, W), 1).astype(jnp.float32)

    trivial = [d for d in range(D) if d == 0 or d == D - 1]
    interior = [d for d in range(D) if 0 < d < D - 1]

    planes = {}
    for d in interior:
        for z in (d - 1, d, d + 1):
            if z not in planes:
                planes[z] = _Plane(x_ref[0, z], need_diag=(z in interior),
                                   need_max8=True, need_max9=True)

    for d in trivial:
        y_ref[0, 0, d] = x_ref[0, d]
        coords_ref[0, 0, 0, d] = jnp.full((H, W), float(d), jnp.float32)
        coords_ref[0, 0, 1, d] = row_f
        coords_ref[0, 0, 2, d] = col_f

    for d in interior:
        c = planes[d]
        lo = planes[d - 1]
        hi = planes[d + 1]
        z0 = c.p

        gx = 0.5 * (c.wp - c.wm)
        gy = 0.5 * (c.hp - c.hm)
        gs = 0.5 * (hi.p - lo.p)
        dxx = c.wp + c.wm - 2.0 * z0
        dyy = c.hp + c.hm - 2.0 * z0
        dss = hi.p + lo.p - 2.0 * z0
        dxy = 0.25 * ((c.d_pp - c.d_pm) - (c.d_mp - c.d_mm))
        dys = 0.25 * ((hi.hp - hi.hm) - (lo.hp - lo.hm))
        dxs = 0.25 * ((hi.wp - hi.wm) - (lo.wp - lo.wm))

        mx = jnp.maximum(jnp.maximum(lo.max9, hi.max9), c.max8)
        mask = z0 > mx

        ha = dxx + n00; hb = dxy + n01; hc = dxs + n02
        hd = dxy + n10; he = dyy + n11; hf = dys + n12
        hg = dxs + n20; hh = dys + n21; hi_ = dss + n22

        A11 = he * hi_ - hf * hh; A12 = hc * hh - hb * hi_; A13 = hb * hf - hc * he
        A21 = hf * hg - hd * hi_; A22 = ha * hi_ - hc * hg; A23 = hc * hd - ha * hf
        A31 = hd * hh - he * hg; A32 = hb * hg - ha * hh; A33 = ha * he - hb * hd
        det = ha * A11 + hb * A21 + hc * A31
        neg_inv_det = -1.0 / det
        dx0 = (A11 * gx + A12 * gy + A13 * gs) * neg_inv_det
        dx1 = (A21 * gx + A22 * gy + A23 * gs) * neg_inv_det
        dx2 = (A31 * gx + A32 * gy + A33 * gs) * neg_inv_det

        big = jnp.maximum(jnp.maximum(jnp.abs(dx0), jnp.abs(dx1)),
                          jnp.abs(dx2)) > 0.7
        keep = mask & jnp.logical_not(big)
        dx0 = jnp.where(keep, dx0, 0.0)
        dx1 = jnp.where(keep, dx1, 0.0)
        dx2 = jnp.where(keep, dx2, 0.0)

        dy_corr = 0.5 * (gx * dx0 + gy * dx1 + gs * dx2)
        y_ref[0, 0, d] = z0 + dy_corr + STRICT_BONUS * mask.astype(jnp.float32)

        coords_ref[0, 0, 0, d] = float(d) + dx2
        coords_ref[0, 0, 1, d] = row_f + dx1
        coords_ref[0, 0, 2, d] = col_f + dx0


@jax.jit
def kernel(x):
    B, C, D, H, W = x.shape
    noise = jnp.abs(jax.random.uniform(jax.random.key(42), (3, 3), dtype=x.dtype)) * NOISE_EPS
    xr = x.reshape(B * C, D, H, W)
    coords, y = pl.pallas_call(
        functools.partial(_stencil_kernel, D=D, H=H, W=W),
        grid=(B * C,),
        out_shape=(
            jax.ShapeDtypeStruct((B, C, 3, D, H, W), x.dtype),
            jax.ShapeDtypeStruct((B, C, D, H, W), x.dtype),
        ),
        in_specs=[
            pl.BlockSpec((1, D, H, W), lambda b: (b, 0, 0, 0)),
            pl.BlockSpec(memory_space=pltpu.SMEM),
        ],
        out_specs=(
            pl.BlockSpec((1, 1, 3, D, H, W), lambda b: (b, 0, 0, 0, 0, 0)),
            pl.BlockSpec((1, 1, D, H, W), lambda b: (b, 0, 0, 0, 0)),
        ),
        compiler_params=pltpu.CompilerParams(
            dimension_semantics=("arbitrary",),
        ),
    )(xr, noise)
    return coords, y
